# baseline (speedup 1.0000x reference)
import jax
import jax.numpy as jnp
from jax import lax
from jax.experimental import pallas as pl
from jax.experimental.pallas import tpu as pltpu
from jax.experimental.pallas import tpu_sc as plsc

VOCAB = 8192
SUB, LANES = 8, VOCAB // 8
ROWS = 64

NC, NS = 2, 16
NW = NC * NS
CH = 2
NBUF = 4


def _sc_gather_body(xf2, table_hbm, out_hbm, idx_v, *rest):
    bufs = rest[:NBUF]
    gsem = rest[NBUF : 2 * NBUF]
    wsem = rest[2 * NBUF : 3 * NBUF]
    n = xf2.shape[0] * xf2.shape[1]
    bpw = n // NW
    nch = bpw // CH

    wid = lax.axis_index("s") * NC + lax.axis_index("c")
    pltpu.sync_copy(xf2.at[pl.ds(wid * nch, nch)], idx_v)

    def gather(k, b):
        return pltpu.make_async_copy(table_hbm.at[idx_v.at[k]], bufs[b], gsem[b])

    def wout(k, b):
        rows = (wid * nch + k) * CH
        return pltpu.make_async_copy(bufs[b], out_hbm.at[pl.ds(rows, CH)], wsem[b])

    for b in range(NBUF):
        gather(b, b).start()

    nrounds = nch // NBUF

    def round_(j, carry):
        for b in range(NBUF):
            k = j * NBUF + b
            gather(k, b).wait()
            wout(k, b).start()
            wout(k, b).wait()
            gather(k + NBUF, b).start()
        return carry

    lax.fori_loop(0, nrounds - 1, round_, 0)
    for b in range(NBUF):
        k = (nrounds - 1) * NBUF + b
        gather(k, b).wait()
        wout(k, b).start()
    for b in range(NBUF):
        wout((nrounds - 1) * NBUF + b, b).wait()


def _loss_body(xb_ref, yb_ref, *refs):
    row_refs = refs[:ROWS]
    yv_ref = refs[ROWS]
    loss_ref = refs[ROWS + 1]
    i = pl.program_id(0)

    @pl.when(i == 0)
    def _():
        loss_ref[...] = jnp.zeros_like(loss_ref)

    rows = jnp.concatenate([r[...] for r in row_refs], axis=0)
    sub = lax.broadcasted_iota(jnp.int32, (ROWS, SUB, LANES), 1)
    lane = lax.broadcasted_iota(jnp.int32, (ROWS, SUB, LANES), 2)
    flat_idx = sub * LANES + lane

    m = jnp.max(rows, axis=(1, 2), keepdims=True)
    s = jnp.sum(jnp.exp(rows - m), axis=(1, 2), keepdims=True)
    lse = m + jnp.log(s)
    y = yv_ref[0, 0][:, None, None]
    tgt = jnp.sum(jnp.where(flat_idx == y, rows, 0.0), axis=(1, 2), keepdims=True)
    loss_ref[...] += jnp.sum(lse - tgt)


def _tc_loss(xf, yf, table3, n):
    grid = (n // ROWS,)

    def row_map(k):
        def index_map(i, xb_ref, yb_ref):
            return (xb_ref[i * ROWS + k], 0, 0)

        return index_map

    in_specs = [pl.BlockSpec((1, SUB, LANES), row_map(k)) for k in range(ROWS)]
    yv = yf.reshape(n // ROWS, 1, ROWS)
    in_specs.append(pl.BlockSpec((1, 1, ROWS), lambda i, xb_ref, yb_ref: (i, 0, 0)))

    grid_spec = pltpu.PrefetchScalarGridSpec(
        num_scalar_prefetch=2,
        grid=grid,
        in_specs=in_specs,
        out_specs=[pl.BlockSpec((1, 1), lambda i, xb_ref, yb_ref: (0, 0))],
    )

    (loss_sum,) = pl.pallas_call(
        _loss_body,
        grid_spec=grid_spec,
        out_shape=[jax.ShapeDtypeStruct((1, 1), jnp.float32)],
        compiler_params=pltpu.CompilerParams(
            dimension_semantics=("arbitrary",),
        ),
    )(xf, yf, *([table3] * ROWS), yv)
    return loss_sum[0, 0] / n


def kernel(xb, yb, table):
    B, T = xb.shape
    N = B * T
    xf = xb.reshape(N).astype(jnp.int32)
    yf = yb.reshape(N).astype(jnp.int32)

    sc_gather = pl.kernel(
        _sc_gather_body,
        out_type=jax.ShapeDtypeStruct((N, VOCAB), jnp.float32),
        mesh=plsc.VectorSubcoreMesh(core_axis_name="c", subcore_axis_name="s"),
        scratch_types=(
            [pltpu.VMEM((N // NW // CH, CH), jnp.int32)]
            + [pltpu.VMEM((CH, VOCAB), jnp.float32)] * NBUF
            + [pltpu.SemaphoreType.DMA] * (2 * NBUF)
        ),
        compiler_params=pltpu.CompilerParams(use_tc_tiling_on_sc=True),
    )
    logits = sc_gather(xf.reshape(N // CH, CH), table)

    table3 = table.reshape(VOCAB, SUB, LANES)
    loss = _tc_loss(xf, yf, table3, N)
    return (logits.reshape(B, T, VOCAB), loss)


# speedup vs baseline: 1.6383x; 1.6383x over previous
import jax
import jax.numpy as jnp
from jax import lax
from jax.experimental import pallas as pl
from jax.experimental.pallas import tpu as pltpu
from jax.experimental.pallas import tpu_sc as plsc

VOCAB = 8192

NC, NS = 2, 16
NW = NC * NS
CH = 2
NBUF = 4
L = 16
UN = 8


def _row_stats(buf, r, y):
    ninf = jnp.full((L,), -jnp.inf, jnp.float32)
    lane = lax.iota(jnp.int32, L)
    y_b = jnp.full((L,), y, jnp.int32)

    def pass_max(j, accs):
        base = j * (L * UN)
        return tuple(
            jnp.maximum(accs[u], buf[r, pl.ds(base + u * L, L)]) for u in range(UN)
        )

    accs = lax.fori_loop(0, VOCAB // (L * UN), pass_max, (ninf,) * UN)
    mv = accs[0]
    for u in range(1, UN):
        mv = jnp.maximum(mv, accs[u])
    m = jnp.max(mv)

    zero = jnp.zeros((L,), jnp.float32)

    def pass_sum(j, carry):
        saccs, tacc = carry
        base = j * (L * UN)
        new_s = []
        for u in range(UN):
            x = buf[r, pl.ds(base + u * L, L)]
            new_s.append(saccs[u] + jnp.exp(x - m))
            tacc = tacc + jnp.where(base + u * L + lane == y_b, x, 0.0)
        return tuple(new_s), tacc

    saccs, tacc = lax.fori_loop(
        0, VOCAB // (L * UN), pass_sum, ((zero,) * UN, zero)
    )
    sv = saccs[0]
    for u in range(1, UN):
        sv = sv + saccs[u]
    s = jnp.sum(sv)
    t = jnp.sum(tacc)
    return m, s, t


def _sc_body(xf2, yf2, table_hbm, out_hbm, s_out, mt_out, idx_v, y_v, s_buf, mt_buf, *rest):
    bufs = rest[:NBUF]
    gsem = rest[NBUF : 2 * NBUF]
    wsem = rest[2 * NBUF : 3 * NBUF]
    n = xf2.shape[0] * xf2.shape[1]
    bpw = n // NW
    nch = bpw // CH

    wid = lax.axis_index("s") * NC + lax.axis_index("c")
    pltpu.sync_copy(xf2.at[pl.ds(wid * nch, nch)], idx_v)
    pltpu.sync_copy(yf2.at[pl.ds(wid * bpw, bpw)], y_v)

    lane0 = lax.iota(jnp.int32, L) == 0

    def gather(k, b):
        return pltpu.make_async_copy(table_hbm.at[idx_v.at[k]], bufs[b], gsem[b])

    def wout(k, b):
        rows = (wid * nch + k) * CH
        return pltpu.make_async_copy(bufs[b], out_hbm.at[pl.ds(rows, CH)], wsem[b])

    lane_i = lax.iota(jnp.int32, L)

    def stats(k, b):
        for r in range(CH):
            sl = k * CH + r
            yg = y_v[pl.ds((sl // L) * L, L)]
            y = jnp.sum(
                jnp.where(lane_i == sl % L, yg.astype(jnp.float32), 0.0)
            ).astype(jnp.int32)
            m, s, t = _row_stats(bufs[b], r, y)
            slot = jnp.full((L,), k * CH + r, jnp.int32)
            plsc.store_scatter(s_buf, [slot], jnp.full((L,), s, jnp.float32), mask=lane0)
            plsc.store_scatter(mt_buf, [slot], jnp.full((L,), m - t, jnp.float32), mask=lane0)

    for b in range(NBUF):
        gather(b, b).start()

    nrounds = nch // NBUF

    def round_(j, carry):
        for b in range(NBUF):
            k = j * NBUF + b
            gather(k, b).wait()
            wout(k, b).start()
            stats(k, b)
            wout(k, b).wait()
            gather(k + NBUF, b).start()
        return carry

    lax.fori_loop(0, nrounds - 1, round_, 0)
    for b in range(NBUF):
        k = (nrounds - 1) * NBUF + b
        gather(k, b).wait()
        wout(k, b).start()
        stats(k, b)
    for b in range(NBUF):
        wout((nrounds - 1) * NBUF + b, b).wait()

    pltpu.sync_copy(s_buf, s_out.at[pl.ds(wid * bpw, bpw)])
    pltpu.sync_copy(mt_buf, mt_out.at[pl.ds(wid * bpw, bpw)])


def _fin_body(s_ref, mt_ref, out_ref):
    n = s_ref.shape[0] * s_ref.shape[1]
    total = jnp.sum(jnp.log(s_ref[...])) + jnp.sum(mt_ref[...])
    out_ref[...] = jnp.reshape(total / n, (1, 1))


def kernel(xb, yb, table):
    B, T = xb.shape
    N = B * T
    xf = xb.reshape(N).astype(jnp.int32)
    yf = yb.reshape(N).astype(jnp.int32)

    sc_call = pl.kernel(
        _sc_body,
        out_type=(
            jax.ShapeDtypeStruct((N, VOCAB), jnp.float32),
            jax.ShapeDtypeStruct((N,), jnp.float32),
            jax.ShapeDtypeStruct((N,), jnp.float32),
        ),
        mesh=plsc.VectorSubcoreMesh(core_axis_name="c", subcore_axis_name="s"),
        scratch_types=(
            [
                pltpu.VMEM((N // NW // CH, CH), jnp.int32),
                pltpu.VMEM((N // NW,), jnp.int32),
                pltpu.VMEM((N // NW,), jnp.float32),
                pltpu.VMEM((N // NW,), jnp.float32),
            ]
            + [pltpu.VMEM((CH, VOCAB), jnp.float32)] * NBUF
            + [pltpu.SemaphoreType.DMA] * (2 * NBUF)
        ),
        compiler_params=pltpu.CompilerParams(needs_layout_passes=False),
    )
    logits, s_arr, mt_arr = sc_call(xf.reshape(N // CH, CH), yf, table)

    (loss,) = pl.pallas_call(
        _fin_body,
        out_shape=[jax.ShapeDtypeStruct((1, 1), jnp.float32)],
    )(s_arr.reshape(8, N // 8), mt_arr.reshape(8, N // 8))

    return (logits.reshape(B, T, VOCAB), loss[0, 0])
